# baseline (speedup 1.0000x reference)
import functools

import jax
import jax.numpy as jnp
from jax import lax
from jax.experimental import pallas as pl
from jax.experimental.pallas import tpu as pltpu
from jax.experimental.pallas import tpu_sc as plsc

_N = 16384
_SCALE = 1000.0
_TAB = 1001
_TAB_PAD = 1008
_NC, _NS, _L = 2, 16, 16
_NW = _NC * _NS
_B_PER_W = _N // _NW

_MAGIC = 12582912.0

_mesh = plsc.VectorSubcoreMesh(core_axis_name="c", subcore_axis_name="s")


@functools.partial(
    pl.kernel,
    out_type=jax.ShapeDtypeStruct((_N,), jnp.float32),
    mesh=_mesh,
    scratch_types=[
        pltpu.VMEM((_B_PER_W,), jnp.float32),
        pltpu.VMEM((_TAB_PAD,), jnp.float32),
        pltpu.VMEM((_B_PER_W,), jnp.float32),
        pltpu.SemaphoreType.DMA,
        pltpu.SemaphoreType.DMA,
    ],
    compiler_params=pltpu.CompilerParams(
        needs_layout_passes=False, skip_device_barrier=True),
)
def _lookup(t_hbm, gamma_hbm, out_hbm, t_v, gamma_v, out_v, sem_t, sem_g):
    wid = lax.axis_index("s") * _NC + lax.axis_index("c")
    base = wid * _B_PER_W
    cp_t = pltpu.make_async_copy(t_hbm.at[pl.ds(base, _B_PER_W)], t_v, sem_t)
    cp_g = pltpu.make_async_copy(
        gamma_hbm, gamma_v.at[pl.ds(0, _TAB)], sem_g)
    cp_t.start()
    cp_g.start()
    cp_t.wait()
    cp_g.wait()
    for i in range(_B_PER_W // _L):
        tv = t_v[pl.ds(i * _L, _L)]
        x = tv * jnp.float32(_SCALE)
        r = (x + jnp.float32(_MAGIC)) - jnp.float32(_MAGIC)
        idx = r.astype(jnp.int32)
        out_v[pl.ds(i * _L, _L)] = plsc.load_gather(gamma_v, [idx])
    pltpu.sync_copy(out_v, out_hbm.at[pl.ds(base, _B_PER_W)])


def kernel(t, gamma):
    return _lookup(t, gamma)


# speedup vs baseline: 1.0231x; 1.0231x over previous
import functools

import jax
import jax.numpy as jnp
from jax import lax
from jax.experimental import pallas as pl
from jax.experimental.pallas import tpu as pltpu
from jax.experimental.pallas import tpu_sc as plsc

_N = 16384
_SCALE = 1000.0
_TAB = 1001
_TAB_PAD = 1008
_NC, _NS, _L = 2, 16, 16
_NW = _NC * _NS
_B_PER_W = _N // _NW

_MAGIC = 12582912.0

_mesh = plsc.VectorSubcoreMesh(core_axis_name="c", subcore_axis_name="s")


@functools.partial(
    pl.kernel,
    out_type=jax.ShapeDtypeStruct((_N,), jnp.float32),
    mesh=_mesh,
    scratch_types=[
        pltpu.VMEM((_B_PER_W,), jnp.float32),
        pltpu.VMEM((_TAB_PAD,), jnp.float32),
        pltpu.VMEM((_B_PER_W,), jnp.float32),
        pltpu.SemaphoreType.DMA,
        pltpu.SemaphoreType.DMA,
    ],
    compiler_params=pltpu.CompilerParams(needs_layout_passes=False),
)
def _lookup(t_hbm, gamma_hbm, out_hbm, t_v, gamma_v, out_v, sem_t, sem_g):
    wid = lax.axis_index("s") * _NC + lax.axis_index("c")
    base = wid * _B_PER_W
    cp_t = pltpu.make_async_copy(t_hbm.at[pl.ds(base, _B_PER_W)], t_v, sem_t)
    cp_g = pltpu.make_async_copy(
        gamma_hbm, gamma_v.at[pl.ds(0, _TAB)], sem_g)
    cp_t.start()
    cp_g.start()
    cp_t.wait()
    cp_g.wait()
    def body(i, carry):
        off = i * _L
        tv = t_v[pl.ds(off, _L)]
        x = tv * jnp.float32(_SCALE)
        r = (x + jnp.float32(_MAGIC)) - jnp.float32(_MAGIC)
        idx = r.astype(jnp.int32)
        out_v[pl.ds(off, _L)] = plsc.load_gather(gamma_v, [idx])
        return carry

    lax.fori_loop(0, _B_PER_W // _L, body, 0)
    pltpu.sync_copy(out_v, out_hbm.at[pl.ds(base, _B_PER_W)])


def kernel(t, gamma):
    return _lookup(t, gamma)


# speedup vs baseline: 1.0256x; 1.0025x over previous
import functools

import jax
import jax.numpy as jnp
from jax import lax
from jax.experimental import pallas as pl
from jax.experimental.pallas import tpu as pltpu
from jax.experimental.pallas import tpu_sc as plsc

_N = 16384
_SCALE = 1000.0
_TAB = 1001
_TAB_PAD = 1008
_NC, _NS, _L = 2, 16, 16
_NW = _NC * _NS
_B_PER_W = _N // _NW

_MAGIC = 12582912.0

_mesh = plsc.VectorSubcoreMesh(core_axis_name="c", subcore_axis_name="s")


@functools.partial(
    pl.kernel,
    out_type=jax.ShapeDtypeStruct((_N,), jnp.float32),
    mesh=_mesh,
    scratch_types=[
        pltpu.VMEM((_B_PER_W,), jnp.float32),
        pltpu.VMEM((_TAB_PAD,), jnp.float32),
        pltpu.VMEM((_B_PER_W,), jnp.float32),
        pltpu.SemaphoreType.DMA,
        pltpu.SemaphoreType.DMA,
    ],
    compiler_params=pltpu.CompilerParams(needs_layout_passes=False),
)
def _lookup(t_hbm, gamma_hbm, out_hbm, t_v, gamma_v, out_v, sem_t, sem_g):
    wid = lax.axis_index("s") * _NC + lax.axis_index("c")
    base = wid * _B_PER_W
    cp_t = pltpu.make_async_copy(t_hbm.at[pl.ds(base, _B_PER_W)], t_v, sem_t)
    cp_g = pltpu.make_async_copy(
        gamma_hbm, gamma_v.at[pl.ds(0, _TAB)], sem_g)
    cp_t.start()
    cp_g.start()
    cp_t.wait()
    cp_g.wait()
    @plsc.parallel_loop(0, _B_PER_W, step=_L, unroll=4)
    def body(off):
        tv = t_v[pl.ds(off, _L)]
        x = tv * jnp.float32(_SCALE)
        r = (x + jnp.float32(_MAGIC)) - jnp.float32(_MAGIC)
        idx = r.astype(jnp.int32)
        out_v[pl.ds(off, _L)] = plsc.load_gather(gamma_v, [idx])

    pltpu.sync_copy(out_v, out_hbm.at[pl.ds(base, _B_PER_W)])


def kernel(t, gamma):
    return _lookup(t, gamma)
